# baseline (speedup 1.0000x reference)
import functools

import jax
import jax.numpy as jnp
from jax.experimental import pallas as pl
from jax.experimental.pallas import tpu as pltpu


def _outer(x_hbm, pgb_ref, o_hbm, *, S, H, tb, nsteps, eps):
    c = pl.program_id(0)
    pos = pgb_ref[:S, :]
    gamma = pgb_ref[S, :]
    beta = pgb_ref[S + 1, :]

    def inner(x_blk, o_blk):
        x = x_blk[...] + pos
        m = jnp.mean(x, axis=-1, keepdims=True)
        m2 = jnp.mean(x * x, axis=-1, keepdims=True)
        var = jnp.maximum(m2 - m * m, 0.0)
        inv = jax.lax.rsqrt(var + jnp.float32(eps))
        o_blk[...] = (x - m) * (inv * gamma) + beta

    pltpu.emit_pipeline(
        inner,
        grid=(nsteps,),
        in_specs=[pl.BlockSpec(
            (tb, S, H), lambda j: (c * nsteps + j, 0, 0),
            pipeline_mode=pl.Buffered(buffer_count=5))],
        out_specs=[pl.BlockSpec(
            (tb, S, H), lambda j: (c * nsteps + j, 0, 0),
            pipeline_mode=pl.Buffered(buffer_count=2))],
    )(x_hbm, o_hbm)


def kernel(input_vis_feats, pos_table, gamma, beta, eps=1e-12):
    B, S, H = input_vis_feats.shape
    pgb = jnp.concatenate(
        [pos_table[:S], gamma.reshape(1, H), beta.reshape(1, H)], axis=0
    )

    itemsize = jnp.dtype(input_vis_feats.dtype).itemsize
    row_bytes = S * H * itemsize
    tb = 1
    while tb < B and B % (tb * 2) == 0 and (tb * 2) * row_bytes <= (3 << 20):
        tb *= 2
    nsteps = B // tb // 2

    return pl.pallas_call(
        functools.partial(_outer, S=S, H=H, tb=tb, nsteps=nsteps, eps=eps),
        out_shape=jax.ShapeDtypeStruct((B, S, H), input_vis_feats.dtype),
        grid=(2,),
        in_specs=[
            pl.BlockSpec(memory_space=pltpu.MemorySpace.HBM),
            pl.BlockSpec((S + 2, H), lambda i: (0, 0)),
        ],
        out_specs=pl.BlockSpec(memory_space=pltpu.MemorySpace.HBM),
        compiler_params=pltpu.CompilerParams(
            dimension_semantics=("parallel",),
            allow_input_fusion=[False, True],
            vmem_limit_bytes=48 << 20,
            skip_device_barrier=True,
        ),
    )(input_vis_feats, pgb)


# speedup vs baseline: 1.0245x; 1.0245x over previous
import functools

import jax
import jax.numpy as jnp
from jax.experimental import pallas as pl
from jax.experimental.pallas import tpu as pltpu


def _outer(x_hbm, pgb_ref, o_hbm, *, S, H, tb, nsteps, eps):
    c = pl.program_id(0)
    pos = pgb_ref[:S, :]
    gamma = pgb_ref[S, :]
    beta = pgb_ref[S + 1, :]

    def inner(x_blk, o_blk):
        x = x_blk[...] + pos
        m = jnp.mean(x, axis=-1, keepdims=True)
        m2 = jnp.mean(x * x, axis=-1, keepdims=True)
        var = jnp.maximum(m2 - m * m, 0.0)
        inv = jax.lax.rsqrt(var + jnp.float32(eps))
        o_blk[...] = (x - m) * (inv * gamma) + beta

    pltpu.emit_pipeline(
        inner,
        grid=(nsteps,),
        in_specs=[pl.BlockSpec(
            (tb, S, H), lambda j: (c * nsteps + j, 0, 0),
            pipeline_mode=pl.Buffered(buffer_count=3, use_lookahead=True))],
        out_specs=[pl.BlockSpec(
            (tb, S, H), lambda j: (c * nsteps + j, 0, 0),
            pipeline_mode=pl.Buffered(buffer_count=2))],
    )(x_hbm, o_hbm)


def kernel(input_vis_feats, pos_table, gamma, beta, eps=1e-12):
    B, S, H = input_vis_feats.shape
    pgb = jnp.concatenate(
        [pos_table[:S], gamma.reshape(1, H), beta.reshape(1, H)], axis=0
    )

    itemsize = jnp.dtype(input_vis_feats.dtype).itemsize
    row_bytes = S * H * itemsize
    tb = 1
    while tb < B and B % (tb * 2) == 0 and (tb * 2) * row_bytes <= (6 << 20):
        tb *= 2
    nsteps = B // tb // 2

    return pl.pallas_call(
        functools.partial(_outer, S=S, H=H, tb=tb, nsteps=nsteps, eps=eps),
        out_shape=jax.ShapeDtypeStruct((B, S, H), input_vis_feats.dtype),
        grid=(2,),
        in_specs=[
            pl.BlockSpec(memory_space=pltpu.MemorySpace.HBM),
            pl.BlockSpec((S + 2, H), lambda i: (0, 0)),
        ],
        out_specs=pl.BlockSpec(memory_space=pltpu.MemorySpace.HBM),
        compiler_params=pltpu.CompilerParams(
            dimension_semantics=("parallel",),
            allow_input_fusion=[False, True],
            vmem_limit_bytes=48 << 20,
            skip_device_barrier=True,
        ),
    )(input_vis_feats, pgb)
